# baseline (speedup 1.0000x reference)
import functools

import jax
import jax.numpy as jnp
from jax import lax
from jax.experimental import pallas as pl
from jax.experimental.pallas import tpu as pltpu
from jax.experimental.pallas import tpu_sc as plsc

_N = 10000
_E = 320000
_D = 128
_DH = _D // 2
_NPAD = 10240
_NC = 2
_NS = 16
_NW = _NC * _NS
_EPW = _E // _NW
_EPT = _E // _NS
_EPT_PAD = 20480
_CHUNK = 128
_NCHUNK = _EPT_PAD // _CHUNK
_GRP = 2
_TRASH = _NPAD - 1
_RPT = _NPAD // _NS
_SB = _RPT // _CHUNK

_MESH = plsc.VectorSubcoreMesh(
    core_axis_name="c", subcore_axis_name="s", num_cores=_NC, num_subcores=_NS
)
_SC_PARAMS = pltpu.CompilerParams(
    needs_layout_passes=False, use_tc_tiling_on_sc=False
)


def _sc_degrees(src2, dst2):
  @functools.partial(
      pl.kernel,
      out_type=jax.ShapeDtypeStruct((2, _NW, _NPAD), jnp.float32),
      mesh=_MESH,
      compiler_params=_SC_PARAMS,
      scratch_types=[
          pltpu.VMEM((_EPW,), jnp.int32),
          pltpu.VMEM((_EPW,), jnp.int32),
          pltpu.VMEM((_NPAD,), jnp.float32),
          pltpu.VMEM((_NPAD,), jnp.float32),
      ],
  )
  def k(src_hbm, dst_hbm, out_hbm, src_v, dst_v, hist_s, hist_d):
    c = lax.axis_index("c")
    s = lax.axis_index("s")
    wid = c * _NS + s
    zero = jnp.zeros((16,), jnp.float32)

    def zb(i, carry):
      hist_s[pl.ds(i * 16, 16)] = zero
      hist_d[pl.ds(i * 16, 16)] = zero
      return carry

    lax.fori_loop(0, _NPAD // 16, zb, 0)
    pltpu.sync_copy(src_hbm.at[wid], src_v)
    pltpu.sync_copy(dst_hbm.at[wid], dst_v)
    ones = jnp.ones((16,), jnp.float32)

    def eb(i, carry):
      plsc.addupdate_scatter(hist_s, [src_v[pl.ds(i * 16, 16)]], ones)
      plsc.addupdate_scatter(hist_d, [dst_v[pl.ds(i * 16, 16)]], ones)
      return carry

    lax.fori_loop(0, _EPW // 16, eb, 0)
    pltpu.sync_copy(hist_s, out_hbm.at[0, wid])
    pltpu.sync_copy(hist_d, out_hbm.at[1, wid])

  return k(src2, dst2)


def _nrsqrt(m):
  i = plsc.bitcast(m, jnp.int32)
  i = 0x5F3759DF - lax.shift_right_logical(i, 1)
  y = plsc.bitcast(i, jnp.float32)
  for _ in range(3):
    y = y * (1.5 - 0.5 * m * y * y)
  return y


def _sc_aggregate(x0, x1, deg_part, sd4):
  @functools.partial(
      pl.kernel,
      out_type=jax.ShapeDtypeStruct((_NC, _NPAD, _DH), jnp.float32),
      mesh=_MESH,
      compiler_params=_SC_PARAMS,
      scratch_types=[
          pltpu.VMEM((2, _GRP, 2, _CHUNK), jnp.int32),
          pltpu.VMEM((_GRP, _CHUNK, _DH), jnp.float32),
          pltpu.VMEM((_NW, _RPT // 2), jnp.float32),
          pltpu.VMEM((_RPT,), jnp.float32),
          pltpu.VMEM_SHARED((_NPAD, _DH), jnp.float32),
          pltpu.VMEM_SHARED((_NPAD, _DH), jnp.float32),
          pltpu.SemaphoreType.DMA,
          pltpu.SemaphoreType.DMA,
          pltpu.SemaphoreType.DMA,
          pltpu.SemaphoreType.DMA,
          pltpu.SemaphoreType.DMA,
          pltpu.SemaphoreType.DMA,
          pltpu.SemaphoreType.DMA,
          pltpu.SemaphoreType.DMA,
          pltpu.SemaphoreType.DMA,
          pltpu.SemaphoreType.DMA,
      ],
  )
  def k(x0_hbm, x1_hbm, deg_hbm, sd_hbm, out_hbm, sdv, rows_v, degv, nrm_v,
        acc, h_sp,
        sem_i0, sem_i1, sem_ga, sem_gb, sem_sa, sem_sb,
        sem_qga, sem_qgb, sem_qsa, sem_qsb):
    c = lax.axis_index("c")
    s = lax.axis_index("s")
    zero = jnp.zeros((16,), jnp.float32)
    nsub = _DH // 16
    half = _GRP // 2
    base = s * _RPT

    pltpu.async_copy(sd_hbm.at[s, pl.ds(0, _GRP)], sdv.at[0], sem_i0)

    def zb(i, carry):
      rows_v[1, i // nsub, pl.ds((i % nsub) * 16, 16)] = zero
      return carry

    lax.fori_loop(0, _CHUNK * nsub, zb, 0)
    for r in range(_SB):
      pltpu.async_copy(rows_v.at[1], acc.at[pl.ds(base + r * _CHUNK, _CHUNK)],
                       sem_sa)

    hrpt = _RPT // 2
    one16 = jnp.ones((16,), jnp.float32)
    for hh in range(2):
      pltpu.sync_copy(
          deg_hbm.at[0, slice(None), pl.ds(base + hh * hrpt, hrpt)], degv)

      def red(i, carry):
        d = jnp.zeros((16,), jnp.float32)
        for r in range(_NW):
          d = d + degv[r, pl.ds(i * 16, 16)]
        nrm_v[pl.ds(hh * hrpt + i * 16, 16)] = _nrsqrt(jnp.maximum(d, one16))
        return carry

      lax.fori_loop(0, hrpt // 16, red, 0)

    def stage(x_hbm):
      def sb(b, carry):
        pltpu.sync_copy(x_hbm.at[pl.ds(base + b * _CHUNK, _CHUNK)],
                        rows_v.at[0])
        for g in range(_CHUNK // 16):
          n16 = nrm_v[pl.ds(b * _CHUNK + g * 16, 16)]
          for i in range(16):
            splat = n16.at[jnp.full((16,), i, jnp.int32)].get(
                mode="promise_in_bounds")
            for q in range(nsub):
              ridx = g * 16 + i
              rows_v[0, ridx, pl.ds(q * 16, 16)] = (
                  rows_v[0, ridx, pl.ds(q * 16, 16)] * splat)
        pltpu.sync_copy(rows_v.at[0],
                        h_sp.at[pl.ds(base + b * _CHUNK, _CHUNK)])
        return carry

      lax.fori_loop(0, _SB, sb, 0)

    @pl.when(c == 0)
    def _():
      stage(x0_hbm)

    @pl.when(c == 1)
    def _():
      stage(x1_hbm)

    for r in range(_SB):
      pltpu.make_async_copy(
          rows_v.at[1], acc.at[pl.ds(base, _CHUNK)], sem_sa).wait()

    plsc.subcore_barrier()

    def fire_gathers(bank, lo, sem):
      return [
          pltpu.async_copy(h_sp.at[sdv.at[bank, lo + i, 0]],
                           rows_v.at[lo + i], sem)
          for i in range(half)
      ]

    def fire_scatters(bank, lo, sem):
      for i in range(half):
        pltpu.async_copy(rows_v.at[lo + i],
                         acc.at[sdv.at[bank, lo + i, 1]], sem, add=True)

    def scat_drain(sem):
      pltpu.make_async_copy(rows_v.at[0], acc.at[sdv.at[0, 0, 1]], sem).wait()

    ngrp2 = _NCHUNK // _GRP // 2

    def body(ii, carry):
      pltpu.make_async_copy(
          sd_hbm.at[s, pl.ds(0, _GRP)], sdv.at[0], sem_i0).wait()

      @pl.when(ii > 0)
      def _():
        for _i in range(half):
          scat_drain(sem_qsa)

      ga = fire_gathers(0, 0, sem_ga)

      @pl.when(ii > 0)
      def _():
        for _i in range(half):
          scat_drain(sem_qsb)

      pltpu.async_copy(
          sd_hbm.at[s, pl.ds((2 * ii + 1) * _GRP, _GRP)], sdv.at[1], sem_i1)
      gb = fire_gathers(0, half, sem_gb)
      for cp in ga:
        cp.wait()
      fire_scatters(0, 0, sem_sa)
      for cp in gb:
        cp.wait()
      fire_scatters(0, half, sem_sb)

      pltpu.make_async_copy(
          sd_hbm.at[s, pl.ds(0, _GRP)], sdv.at[1], sem_i1).wait()
      for _i in range(half):
        scat_drain(sem_sa)
      qga = fire_gathers(1, 0, sem_qga)
      for _i in range(half):
        scat_drain(sem_sb)

      @pl.when(ii < ngrp2 - 1)
      def _():
        pltpu.async_copy(
            sd_hbm.at[s, pl.ds((2 * ii + 2) * _GRP, _GRP)], sdv.at[0], sem_i0)

      qgb = fire_gathers(1, half, sem_qgb)
      for cp in qga:
        cp.wait()
      fire_scatters(1, 0, sem_qsa)
      for cp in qgb:
        cp.wait()
      fire_scatters(1, half, sem_qsb)
      return carry

    lax.fori_loop(0, ngrp2, body, 0)
    for _i in range(half):
      scat_drain(sem_qsa)
    for _i in range(half):
      scat_drain(sem_qsb)

    plsc.subcore_barrier()
    pltpu.sync_copy(acc.at[pl.ds(base, _RPT)], out_hbm.at[c, pl.ds(base, _RPT)])

  return k(x0, x1, deg_part, sd4)


def _tc_final(deg_part, a0, a1, W, b2):
  def body(deg_ref, a0_ref, a1_ref, w_ref, b_ref, o_ref):
    deg = jnp.sum(deg_ref[0], axis=0)
    nd = lax.rsqrt(jnp.maximum(deg, 1.0))
    ndc = jnp.transpose(nd.reshape(1, -1), (1, 0))
    o_ref[...] = (
        jnp.dot(a0_ref[...] * ndc, w_ref[0:_DH, :],
                preferred_element_type=jnp.float32)
        + jnp.dot(a1_ref[...] * ndc, w_ref[_DH:_D, :],
                  preferred_element_type=jnp.float32)
        + b_ref[...]
    )

  rows = 2048
  return pl.pallas_call(
      body,
      grid=(_NPAD // rows,),
      in_specs=[
          pl.BlockSpec((1, _NW, rows), lambda i: (1, 0, i)),
          pl.BlockSpec((rows, _DH), lambda i: (i, 0)),
          pl.BlockSpec((rows, _DH), lambda i: (i, 0)),
          pl.BlockSpec((_D, _D), lambda i: (0, 0)),
          pl.BlockSpec((1, _D), lambda i: (0, 0)),
      ],
      out_specs=pl.BlockSpec((rows, _D), lambda i: (i, 0)),
      out_shape=jax.ShapeDtypeStruct((_NPAD, _D), jnp.float32),
  )(deg_part, a0, a1, W, b2)


def kernel(x, edge_index, W, b):
  src = edge_index[0]
  dst = edge_index[1]
  deg_part = _sc_degrees(src.reshape(_NW, _EPW), dst.reshape(_NW, _EPW))
  xp = jnp.pad(x, ((0, _NPAD - _N), (0, 0)))
  pad = _EPT_PAD - _EPT
  src_p = jnp.pad(src.reshape(_NS, _EPT), ((0, 0), (0, pad)),
                  constant_values=0)
  dst_p = jnp.pad(dst.reshape(_NS, _EPT), ((0, 0), (0, pad)),
                  constant_values=_TRASH)
  sd4 = jnp.stack(
      [src_p.reshape(_NS, _NCHUNK, _CHUNK),
       dst_p.reshape(_NS, _NCHUNK, _CHUNK)],
      axis=2,
  )
  agg = _sc_aggregate(xp[:, 0:_DH], xp[:, _DH:_D], deg_part, sd4)
  out = _tc_final(deg_part, agg[0], agg[1], W, b.reshape(1, _D))
  return out[:_N]
